# Initial kernel scaffold; baseline (speedup 1.0000x reference)
import jax
import jax.numpy as jnp
from jax.experimental import pallas as pl


def kernel(hidden_states, gate_w, w1, w2):
    raise NotImplementedError("write your pallas kernel here")


# speedup vs baseline: 1.4802x; 1.4802x over previous
import functools

import jax
import jax.numpy as jnp
from jax.experimental import pallas as pl
from jax.experimental.pallas import tpu as pltpu


def _dot_t(a, b):
    return jax.lax.dot_general(a, b, (((1,), (1,)), ((), ())),
                               preferred_element_type=jnp.float32)


def _moe_dense_kernel(x_ref, gate_ref, w1g_ref, w1u_ref, w2_ref, out_ref,
                      we_ref):
    e = pl.program_id(0)
    it = pl.program_id(1)

    @pl.when((e == 0) & (it == 0))
    def _router():
        x = x_ref[...]
        logits = _dot_t(x, gate_ref[...])
        m = jnp.max(logits, axis=-1, keepdims=True)
        ex = jnp.exp(logits - m)
        probs = ex / jnp.sum(ex, axis=-1, keepdims=True)
        n_e = probs.shape[-1]
        lane = jax.lax.broadcasted_iota(jnp.int32, probs.shape, 1)
        v1 = jnp.max(probs, axis=-1, keepdims=True)
        i1 = jnp.min(jnp.where(probs == v1, lane, n_e), axis=-1, keepdims=True)
        m1 = lane == i1
        probs2 = jnp.where(m1, -1.0, probs)
        v2 = jnp.max(probs2, axis=-1, keepdims=True)
        i2 = jnp.min(jnp.where(probs2 == v2, lane, n_e), axis=-1,
                     keepdims=True)
        we_ref[...] = jnp.where(m1 | (lane == i2), probs, 0.0)
        out_ref[...] = jnp.zeros_like(out_ref)

    x = x_ref[...]
    g = _dot_t(x, w1g_ref[0])
    u = _dot_t(x, w1u_ref[0])
    act = g * jax.lax.logistic(g) * u
    part = _dot_t(act, w2_ref[0])

    we = we_ref[...]
    colmask = jax.lax.broadcasted_iota(jnp.int32, we.shape, 1) == e
    wcol = jnp.sum(jnp.where(colmask, we, 0.0), axis=-1, keepdims=True)
    out_ref[...] += part * wcol


def kernel(hidden_states, gate_w, w1, w2):
    num_tokens, hidden = hidden_states.shape
    n_experts, two_inter, _ = w1.shape
    inter = two_inter // 2
    tile_i = min(512, inter)
    n_it = inter // tile_i

    grid = (n_experts, n_it)
    out = pl.pallas_call(
        _moe_dense_kernel,
        grid=grid,
        in_specs=[
            pl.BlockSpec((num_tokens, hidden), lambda e, it: (0, 0)),
            pl.BlockSpec((n_experts, hidden), lambda e, it: (0, 0)),
            pl.BlockSpec((1, tile_i, hidden), lambda e, it: (e, it, 0)),
            pl.BlockSpec((1, tile_i, hidden),
                         lambda e, it, n_it=n_it: (e, n_it + it, 0)),
            pl.BlockSpec((1, hidden, tile_i), lambda e, it: (e, 0, it)),
        ],
        out_specs=pl.BlockSpec((num_tokens, hidden), lambda e, it: (0, 0)),
        out_shape=jax.ShapeDtypeStruct((num_tokens, hidden), jnp.float32),
        scratch_shapes=[pltpu.VMEM((num_tokens, n_experts), jnp.float32)],
    )(hidden_states, gate_w, w1, w1, w2)
    return out
